# baseline (device time: 15833 ns/iter reference)
import jax
import jax.numpy as jnp
from jax import lax
from jax.experimental import pallas as pl
from jax.experimental.pallas import tpu as pltpu

_MESH = pl.DeviceIdType.MESH


def kernel(x):
    m, n = x.shape
    half = n // 2
    qrows = m // 4

    def body(x_ref, out_ref, loc_sem,
             xs_sems, xr_sems, ys_sem, yr_sem, zs_sem, zr_sem):
        mx = lax.axis_index("x")
        my = lax.axis_index("y")
        mz = lax.axis_index("z")
        xp = (1 - mx, my, mz)
        yp = (mx, 1 - my, mz)
        zp = (mx, my, 1 - mz)

        barrier = pltpu.get_barrier_semaphore()
        for nbr in (xp, yp, zp):
            pl.semaphore_signal(barrier, inc=1, device_id=nbr,
                                device_id_type=_MESH)

        loc = pltpu.make_async_copy(
            x_ref.at[:, pl.ds(mx * half, half)],
            out_ref.at[pl.ds(mx * m, m), :],
            loc_sem,
        )
        loc.start()

        pl.semaphore_wait(barrier, 3)

        base_out = (1 - mx) * m
        peer_cols = (1 - mx) * half
        qi_me = 2 * my + mz
        qi_dg = 2 * (1 - my) + (1 - mz)

        x_rdmas = []
        for c, src_q in enumerate((qi_me, qi_dg)):
            roff = src_q * qrows
            r = pltpu.make_async_remote_copy(
                src_ref=x_ref.at[pl.ds(roff, qrows), pl.ds(peer_cols, half)],
                dst_ref=out_ref.at[pl.ds(mx * m + roff, qrows), :],
                send_sem=xs_sems.at[c],
                recv_sem=xr_sems.at[c],
                device_id=xp,
                device_id_type=_MESH,
            )
            r.start()
            x_rdmas.append(r)

        x_rdmas[0].wait_recv()
        rows = pl.ds(base_out + qi_me * qrows, qrows)
        yz_rdmas = []
        for partner, ssem, rsem in ((yp, ys_sem, yr_sem),
                                    (zp, zs_sem, zr_sem)):
            r = pltpu.make_async_remote_copy(
                src_ref=out_ref.at[rows, :],
                dst_ref=out_ref.at[rows, :],
                send_sem=ssem,
                recv_sem=rsem,
                device_id=partner,
                device_id_type=_MESH,
            )
            r.start()
            yz_rdmas.append(r)

        x_rdmas[1].wait_recv()
        for r in yz_rdmas:
            r.wait_recv()
        for r in x_rdmas + yz_rdmas:
            r.wait_send()
        loc.wait()

    return pl.pallas_call(
        body,
        out_shape=jax.ShapeDtypeStruct((2 * m, half), x.dtype),
        in_specs=[pl.BlockSpec(memory_space=pltpu.VMEM)],
        out_specs=pl.BlockSpec(memory_space=pltpu.VMEM),
        scratch_shapes=[
            pltpu.SemaphoreType.DMA,
            pltpu.SemaphoreType.DMA((2,)),
            pltpu.SemaphoreType.DMA((2,)),
            pltpu.SemaphoreType.DMA,
            pltpu.SemaphoreType.DMA,
            pltpu.SemaphoreType.DMA,
            pltpu.SemaphoreType.DMA,
        ],
        compiler_params=pltpu.CompilerParams(collective_id=0),
    )(x)


# device time: 13029 ns/iter; 1.2152x vs baseline; 1.2152x over previous
import jax
from jax import lax
from jax.experimental import pallas as pl
from jax.experimental.pallas import tpu as pltpu

_MESH = pl.DeviceIdType.MESH


def kernel(x):
    m, n = x.shape
    half = n // 2
    qrows = m // 4
    nch = 4
    ch = qrows // nch

    def body(x_ref, out_ref, loc_sem,
             xs_sems, xr_sems, ys_sems, yr_sems, zs_sems, zr_sems):
        mx = lax.axis_index("x")
        my = lax.axis_index("y")
        mz = lax.axis_index("z")
        xp = (1 - mx, my, mz)
        yp = (mx, 1 - my, mz)
        zp = (mx, my, 1 - mz)

        barrier = pltpu.get_barrier_semaphore()
        pl.semaphore_signal(barrier, inc=3, device_id=xp,
                            device_id_type=_MESH)
        for nbr in (yp, zp):
            pl.semaphore_signal(barrier, inc=1, device_id=nbr,
                                device_id_type=_MESH)
        loc = pltpu.make_async_copy(
            x_ref.at[:, pl.ds(mx * half, half)],
            out_ref.at[pl.ds(mx * m, m), :],
            loc_sem,
        )
        loc.start()

        pl.semaphore_wait(barrier, 3)

        base_out = (1 - mx) * m
        peer_cols = (1 - mx) * half
        qi_me = 2 * my + mz
        qi_dg = 2 * (1 - my) + (1 - mz)

        x_pieces = [(qi_me * qrows + c * ch, ch) for c in range(nch)]
        x_pieces += [(qi_dg * qrows, qrows // 2),
                     (qi_dg * qrows + qrows // 2, qrows // 2)]
        x_rdmas = []
        for c, (roff, rlen) in enumerate(x_pieces):
            r = pltpu.make_async_remote_copy(
                src_ref=x_ref.at[pl.ds(roff, rlen), pl.ds(peer_cols, half)],
                dst_ref=out_ref.at[pl.ds(mx * m + roff, rlen), :],
                send_sem=xs_sems.at[c],
                recv_sem=xr_sems.at[c],
                device_id=xp,
                device_id_type=_MESH,
            )
            r.start()
            x_rdmas.append(r)

        pl.semaphore_wait(barrier, 2)

        y_rdmas = []
        z_rdmas = []
        for c in range(nch):
            x_rdmas[c].wait_recv()
            rows = pl.ds(base_out + qi_me * qrows + c * ch, ch)
            for partner, ssems, rsems, acc in (
                (yp, ys_sems, yr_sems, y_rdmas),
                (zp, zs_sems, zr_sems, z_rdmas),
            ):
                r = pltpu.make_async_remote_copy(
                    src_ref=out_ref.at[rows, :],
                    dst_ref=out_ref.at[rows, :],
                    send_sem=ssems.at[c],
                    recv_sem=rsems.at[c],
                    device_id=partner,
                    device_id_type=_MESH,
                )
                r.start()
                acc.append(r)

        for c in range(nch, nch + 2):
            x_rdmas[c].wait_recv()
        for c in range(nch):
            y_rdmas[c].wait_recv()
            z_rdmas[c].wait_recv()
        for r in x_rdmas + y_rdmas + z_rdmas:
            r.wait_send()
        loc.wait()

    return pl.pallas_call(
        body,
        out_shape=jax.ShapeDtypeStruct((2 * m, half), x.dtype),
        in_specs=[pl.BlockSpec(memory_space=pltpu.VMEM)],
        out_specs=pl.BlockSpec(memory_space=pltpu.VMEM),
        scratch_shapes=[
            pltpu.SemaphoreType.DMA,
            pltpu.SemaphoreType.DMA((nch + 2,)),
            pltpu.SemaphoreType.DMA((nch + 2,)),
            pltpu.SemaphoreType.DMA((nch,)),
            pltpu.SemaphoreType.DMA((nch,)),
            pltpu.SemaphoreType.DMA((nch,)),
            pltpu.SemaphoreType.DMA((nch,)),
        ],
        compiler_params=pltpu.CompilerParams(collective_id=0),
    )(x)
